# Initial kernel scaffold; baseline (speedup 1.0000x reference)
import jax
import jax.numpy as jnp
from jax.experimental import pallas as pl


def kernel(x, edge_index, batch, W1a, b1a, g1a, be1a, W2a, b2a, g2a, be2a, W1b, b1b, g1b, be1b, W2b, b2b, g2b, be2b, gbn, bbn, Wl, bl):
    raise NotImplementedError("write your pallas kernel here")


# speedup vs baseline: 2.9500x; 2.9500x over previous
import functools

import jax
import jax.numpy as jnp
from jax import lax
from jax.experimental import pallas as pl
from jax.experimental.pallas import tpu as pltpu
from jax.experimental.pallas import tpu_sc as plsc

N = 10000
E = 320000
D = 128
G = 64
BN_EPS = 1e-5

NW = 32
CH = 128
EPT = 10240
NCH = EPT // CH
PAD_E = NW * EPT - E
ROWS_ACC = 10240
STRIPE = ROWS_ACC // 16

BLK = 2000
NBLK = N // BLK


_sc_mesh = plsc.VectorSubcoreMesh(core_axis_name="c", subcore_axis_name="s")


@functools.partial(
    pl.kernel,
    out_type=jax.ShapeDtypeStruct((2, ROWS_ACC, D), jnp.float32),
    mesh=_sc_mesh,
    scratch_types=[
        pltpu.VMEM((NCH, CH), jnp.int32),
        pltpu.VMEM((NCH, CH), jnp.int32),
        pltpu.VMEM((CH, D), jnp.float32),
        pltpu.VMEM_SHARED((ROWS_ACC, D), jnp.float32),
        pltpu.SemaphoreType.DMA,
    ],
)
def _sc_segment_sum(table, srcs, dsts, zeros, out, src_v, dst_v, rows_v, acc, sem):
    c = lax.axis_index("c")
    si = lax.axis_index("s")
    wid = si * 2 + c
    pltpu.sync_copy(zeros, acc.at[pl.ds(si * STRIPE, STRIPE)])
    pltpu.sync_copy(srcs.at[wid], src_v)
    pltpu.sync_copy(dsts.at[wid], dst_v)
    plsc.subcore_barrier()

    def body(j, carry):
        pltpu.async_copy(table.at[src_v.at[j]], rows_v, sem).wait()
        pltpu.sync_copy(rows_v, acc.at[dst_v.at[j]], add=True)
        return carry

    lax.fori_loop(0, NCH, body, 0)
    plsc.subcore_barrier()
    pltpu.sync_copy(acc.at[pl.ds(si * STRIPE, STRIPE)],
                    out.at[c, pl.ds(si * STRIPE, STRIPE)])


def _tc1_body(x_ref, a0_ref, a1_ref, b_ref, w1_ref, b1_ref, w2_ref, b2_ref,
              s2_ref, t2_ref, h_ref, p0_ref, p1_ref):
    i = pl.program_id(0)
    xb = x_ref[...]
    s = xb + a0_ref[...] + a1_ref[...]
    r = jnp.maximum(jnp.dot(s, w1_ref[...], preferred_element_type=jnp.float32)
                    + b1_ref[...], 0.0)
    r = jnp.maximum(jnp.dot(r, w2_ref[...], preferred_element_type=jnp.float32)
                    + b2_ref[...], 0.0)
    h = r * s2_ref[...] + t2_ref[...]
    h_ref[...] = h
    bvec = b_ref[0, 0, :]
    gi = lax.broadcasted_iota(jnp.int32, (G, BLK), 0)
    onehot = jnp.where(bvec[None, :] == gi, 1.0, 0.0)

    @pl.when(i == 0)
    def _():
        p0_ref[...] = jnp.zeros_like(p0_ref)
        p1_ref[...] = jnp.zeros_like(p1_ref)

    p0_ref[...] += jnp.dot(onehot, xb, preferred_element_type=jnp.float32)
    p1_ref[...] += jnp.dot(onehot, h, preferred_element_type=jnp.float32)


def _tc2_body(h_ref, a0_ref, a1_ref, b_ref, w1_ref, b1_ref, w2_ref, b2_ref,
              s2_ref, t2_ref, p0_ref, p1_ref, v0_ref, v1_ref, v2_ref, cst_ref,
              out_ref, p2_acc, cnt_acc):
    i = pl.program_id(0)
    hb = h_ref[...]
    s = hb + a0_ref[...] + a1_ref[...]
    r = jnp.maximum(jnp.dot(s, w1_ref[...], preferred_element_type=jnp.float32)
                    + b1_ref[...], 0.0)
    r = jnp.maximum(jnp.dot(r, w2_ref[...], preferred_element_type=jnp.float32)
                    + b2_ref[...], 0.0)
    h2 = r * s2_ref[...] + t2_ref[...]
    bvec = b_ref[0, 0, :]
    gi = lax.broadcasted_iota(jnp.int32, (G, BLK), 0)
    onehot = jnp.where(bvec[None, :] == gi, 1.0, 0.0)

    @pl.when(i == 0)
    def _():
        p2_acc[...] = jnp.zeros_like(p2_acc)
        cnt_acc[...] = jnp.zeros_like(cnt_acc)

    p2_acc[...] += jnp.dot(onehot, h2, preferred_element_type=jnp.float32)
    cnt_acc[...] += jnp.broadcast_to(
        jnp.sum(onehot, axis=1, keepdims=True), (G, D))

    @pl.when(i == NBLK - 1)
    def _():
        cnt = jnp.maximum(cnt_acc[...], 1.0)
        z = (p0_ref[...] / cnt) * v0_ref[...] \
            + (p1_ref[...] / cnt) * v1_ref[...] \
            + (p2_acc[...] / cnt) * v2_ref[...]
        res = jnp.sum(z, axis=1, keepdims=True) + cst_ref[0, 0]
        out_ref[...] = jnp.broadcast_to(res, (G, D))


def _row_spec(i):
    return (i, 0)


def _full_spec(i):
    return (0, 0)


_W_SPEC = pl.BlockSpec((D, D), _full_spec)
_V_SPEC = pl.BlockSpec((1, D), _full_spec)
_P_SPEC = pl.BlockSpec((G, D), _full_spec)
_X_SPEC = pl.BlockSpec((BLK, D), _row_spec)
_B_SPEC = pl.BlockSpec((1, 1, BLK), lambda i: (i, 0, 0))


def _tc1(x, a0, a1, batch3, w1, b1, w2, b2, s2, t2):
    return pl.pallas_call(
        _tc1_body,
        grid=(NBLK,),
        in_specs=[_X_SPEC, _X_SPEC, _X_SPEC, _B_SPEC,
                  _W_SPEC, _V_SPEC, _W_SPEC, _V_SPEC, _V_SPEC, _V_SPEC],
        out_specs=[_X_SPEC, _P_SPEC, _P_SPEC],
        out_shape=[
            jax.ShapeDtypeStruct((N, D), jnp.float32),
            jax.ShapeDtypeStruct((G, D), jnp.float32),
            jax.ShapeDtypeStruct((G, D), jnp.float32),
        ],
    )(x, a0, a1, batch3, w1, b1, w2, b2, s2, t2)


def _tc2(h1, a0, a1, batch3, w1, b1, w2, b2, s2, t2, p0, p1, v0, v1, v2, cst):
    return pl.pallas_call(
        _tc2_body,
        grid=(NBLK,),
        in_specs=[_X_SPEC, _X_SPEC, _X_SPEC, _B_SPEC,
                  _W_SPEC, _V_SPEC, _W_SPEC, _V_SPEC, _V_SPEC, _V_SPEC,
                  _P_SPEC, _P_SPEC, _V_SPEC, _V_SPEC, _V_SPEC, _V_SPEC],
        out_specs=[_P_SPEC],
        out_shape=[jax.ShapeDtypeStruct((G, D), jnp.float32)],
        scratch_shapes=[
            pltpu.VMEM((G, D), jnp.float32),
            pltpu.VMEM((G, D), jnp.float32),
        ],
    )(h1, a0, a1, batch3, w1, b1, w2, b2, s2, t2, p0, p1, v0, v1, v2, cst)


def kernel(x, edge_index, batch, W1a, b1a, g1a, be1a, W2a, b2a, g2a, be2a,
           W1b, b1b, g1b, be1b, W2b, b2b, g2b, be2b, gbn, bbn, Wl, bl):
    f32 = jnp.float32
    src = edge_index[0]
    dst = edge_index[1]
    pad_src = jnp.zeros((PAD_E,), jnp.int32)
    pad_dst = N + (jnp.arange(PAD_E, dtype=jnp.int32) % 16)
    srcs = jnp.concatenate([src, pad_src]).reshape(NW, NCH, CH)
    dsts = jnp.concatenate([dst, pad_dst]).reshape(NW, NCH, CH)
    zeros = jnp.zeros((STRIPE, D), f32)
    batch3 = batch.reshape(NBLK, 1, BLK)

    sc = 1.0 / jnp.sqrt(1.0 + BN_EPS)
    w2a_f = (g1a * sc)[:, None] * W2a
    b2a_f = (b2a + be1a @ W2a).reshape(1, D)
    s2a = (g2a * sc).reshape(1, D)
    t2a = be2a.reshape(1, D)
    w2b_f = (g1b * sc)[:, None] * W2b
    b2b_f = (b2b + be1b @ W2b).reshape(1, D)
    s2b = (g2b * sc).reshape(1, D)
    t2b = be2b.reshape(1, D)
    wl = Wl[:, 0]
    v = (gbn * sc) * wl
    v0 = v[0:D].reshape(1, D)
    v1 = v[D:2 * D].reshape(1, D)
    v2 = v[2 * D:3 * D].reshape(1, D)
    cst = jnp.full((1, D), bl[0] + bbn @ wl, f32)

    agg1 = _sc_segment_sum(x, srcs, dsts, zeros)
    h1, p0, p1 = _tc1(x, agg1[0, :N], agg1[1, :N], batch3,
                      W1a, b1a.reshape(1, D), w2a_f, b2a_f, s2a, t2a)
    agg2 = _sc_segment_sum(h1, srcs, dsts, zeros)
    outb = _tc2(h1, agg2[0, :N], agg2[1, :N], batch3,
                W1b, b1b.reshape(1, D), w2b_f, b2b_f, s2b, t2b,
                p0, p1, v0, v1, v2, cst)
    return outb[0][:, 0]


# speedup vs baseline: 3.4537x; 1.1708x over previous
import functools

import jax
import jax.numpy as jnp
from jax import lax
from jax.experimental import pallas as pl
from jax.experimental.pallas import tpu as pltpu
from jax.experimental.pallas import tpu_sc as plsc

N = 10000
E = 320000
D = 128
G = 64
BN_EPS = 1e-5

NW = 32
CH = 128
EPT = 10240
NCH = EPT // CH
PAD_E = NW * EPT - E
ROWS_ACC = 10240
STRIPE = ROWS_ACC // 16

BLK = 2000
NBLK = N // BLK


_sc_mesh = plsc.VectorSubcoreMesh(core_axis_name="c", subcore_axis_name="s")


NB = 2
GROUPS = NCH // NB


def _unpack_chunk(pk_v, usrc, udst, j, b):
    for k in range(CH // 16):
        v = pk_v[j, pl.ds(k * 16, 16)]
        usrc[b, pl.ds(k * 16, 16)] = lax.bitwise_and(v, 0x3FFF)
        udst[b, pl.ds(k * 16, 16)] = lax.shift_right_logical(v, 14)


@functools.partial(
    pl.kernel,
    out_type=jax.ShapeDtypeStruct((2, ROWS_ACC, D), jnp.float32),
    mesh=_sc_mesh,
    scratch_types=[
        pltpu.VMEM((NCH, CH), jnp.int32),
        pltpu.VMEM((NB, CH), jnp.int32),
        pltpu.VMEM((NB, CH), jnp.int32),
    ] + [pltpu.VMEM((CH, D), jnp.float32)] * NB
    + [
        pltpu.VMEM_SHARED((ROWS_ACC, D), jnp.float32),
    ] + [pltpu.SemaphoreType.DMA] * (2 * NB),
)
def _sc_segment_sum(table, packed, zeros, out, pk_v, usrc, udst, *rest):
    bufs = rest[:NB]
    acc = rest[NB]
    gsem = rest[NB + 1:NB + 1 + NB]
    ssem = rest[NB + 1 + NB:]
    c = lax.axis_index("c")
    si = lax.axis_index("s")
    wid = si * 2 + c
    pltpu.sync_copy(zeros, acc.at[pl.ds(si * STRIPE, STRIPE)])
    pltpu.sync_copy(packed.at[wid], pk_v)
    plsc.subcore_barrier()

    for b in range(NB):
        _unpack_chunk(pk_v, usrc, udst, b, b)
        pltpu.async_copy(table.at[usrc.at[b]], bufs[b], gsem[b])

    def group(g, carry):
        base = g * NB
        for b in range(NB):
            j = base + b
            pltpu.make_async_copy(table.at[usrc.at[b]], bufs[b],
                                  gsem[b]).wait()
            pltpu.async_copy(bufs[b], acc.at[udst.at[b]], ssem[b],
                             add=True)
        for b in range(NB):
            j = base + b
            pltpu.make_async_copy(bufs[b], acc.at[udst.at[b]],
                                  ssem[b]).wait()

            @pl.when(g < GROUPS - 1)
            def _():
                _unpack_chunk(pk_v, usrc, udst, j + NB, b)
                pltpu.async_copy(table.at[usrc.at[b]], bufs[b], gsem[b])

        return carry

    lax.fori_loop(0, GROUPS, group, 0)
    plsc.subcore_barrier()
    pltpu.sync_copy(acc.at[pl.ds(si * STRIPE, STRIPE)],
                    out.at[c, pl.ds(si * STRIPE, STRIPE)])


def _tc1_body(x_ref, a0_ref, a1_ref, b_ref, w1_ref, b1_ref, w2_ref, b2_ref,
              s2_ref, t2_ref, h_ref, p0_ref, p1_ref):
    i = pl.program_id(0)
    xb = x_ref[...]
    s = xb + a0_ref[...] + a1_ref[...]
    r = jnp.maximum(jnp.dot(s, w1_ref[...], preferred_element_type=jnp.float32)
                    + b1_ref[...], 0.0)
    r = jnp.maximum(jnp.dot(r, w2_ref[...], preferred_element_type=jnp.float32)
                    + b2_ref[...], 0.0)
    h = r * s2_ref[...] + t2_ref[...]
    h_ref[...] = h
    bvec = b_ref[0, 0, :]
    gi = lax.broadcasted_iota(jnp.int32, (G, BLK), 0)
    onehot = jnp.where(bvec[None, :] == gi, 1.0, 0.0)

    @pl.when(i == 0)
    def _():
        p0_ref[...] = jnp.zeros_like(p0_ref)
        p1_ref[...] = jnp.zeros_like(p1_ref)

    p0_ref[...] += jnp.dot(onehot, xb, preferred_element_type=jnp.float32)
    p1_ref[...] += jnp.dot(onehot, h, preferred_element_type=jnp.float32)


def _tc2_body(h_ref, a0_ref, a1_ref, b_ref, w1_ref, b1_ref, w2_ref, b2_ref,
              s2_ref, t2_ref, p0_ref, p1_ref, v0_ref, v1_ref, v2_ref, cst_ref,
              out_ref, p2_acc, cnt_acc):
    i = pl.program_id(0)
    hb = h_ref[...]
    s = hb + a0_ref[...] + a1_ref[...]
    r = jnp.maximum(jnp.dot(s, w1_ref[...], preferred_element_type=jnp.float32)
                    + b1_ref[...], 0.0)
    r = jnp.maximum(jnp.dot(r, w2_ref[...], preferred_element_type=jnp.float32)
                    + b2_ref[...], 0.0)
    h2 = r * s2_ref[...] + t2_ref[...]
    bvec = b_ref[0, 0, :]
    gi = lax.broadcasted_iota(jnp.int32, (G, BLK), 0)
    onehot = jnp.where(bvec[None, :] == gi, 1.0, 0.0)

    @pl.when(i == 0)
    def _():
        p2_acc[...] = jnp.zeros_like(p2_acc)
        cnt_acc[...] = jnp.zeros_like(cnt_acc)

    p2_acc[...] += jnp.dot(onehot, h2, preferred_element_type=jnp.float32)
    cnt_acc[...] += jnp.broadcast_to(
        jnp.sum(onehot, axis=1, keepdims=True), (G, D))

    @pl.when(i == NBLK - 1)
    def _():
        cnt = jnp.maximum(cnt_acc[...], 1.0)
        z = (p0_ref[...] / cnt) * v0_ref[...] \
            + (p1_ref[...] / cnt) * v1_ref[...] \
            + (p2_acc[...] / cnt) * v2_ref[...]
        res = jnp.sum(z, axis=1, keepdims=True) + cst_ref[0, 0]
        out_ref[...] = jnp.broadcast_to(res, (G, D))


def _row_spec(i):
    return (i, 0)


def _full_spec(i):
    return (0, 0)


_W_SPEC = pl.BlockSpec((D, D), _full_spec)
_V_SPEC = pl.BlockSpec((1, D), _full_spec)
_P_SPEC = pl.BlockSpec((G, D), _full_spec)
_X_SPEC = pl.BlockSpec((BLK, D), _row_spec)
_B_SPEC = pl.BlockSpec((1, 1, BLK), lambda i: (i, 0, 0))


def _tc1(x, a0, a1, batch3, w1, b1, w2, b2, s2, t2):
    return pl.pallas_call(
        _tc1_body,
        grid=(NBLK,),
        in_specs=[_X_SPEC, _X_SPEC, _X_SPEC, _B_SPEC,
                  _W_SPEC, _V_SPEC, _W_SPEC, _V_SPEC, _V_SPEC, _V_SPEC],
        out_specs=[_X_SPEC, _P_SPEC, _P_SPEC],
        out_shape=[
            jax.ShapeDtypeStruct((N, D), jnp.float32),
            jax.ShapeDtypeStruct((G, D), jnp.float32),
            jax.ShapeDtypeStruct((G, D), jnp.float32),
        ],
    )(x, a0, a1, batch3, w1, b1, w2, b2, s2, t2)


def _tc2(h1, a0, a1, batch3, w1, b1, w2, b2, s2, t2, p0, p1, v0, v1, v2, cst):
    return pl.pallas_call(
        _tc2_body,
        grid=(NBLK,),
        in_specs=[_X_SPEC, _X_SPEC, _X_SPEC, _B_SPEC,
                  _W_SPEC, _V_SPEC, _W_SPEC, _V_SPEC, _V_SPEC, _V_SPEC,
                  _P_SPEC, _P_SPEC, _V_SPEC, _V_SPEC, _V_SPEC, _V_SPEC],
        out_specs=[_P_SPEC],
        out_shape=[jax.ShapeDtypeStruct((G, D), jnp.float32)],
        scratch_shapes=[
            pltpu.VMEM((G, D), jnp.float32),
            pltpu.VMEM((G, D), jnp.float32),
        ],
    )(h1, a0, a1, batch3, w1, b1, w2, b2, s2, t2, p0, p1, v0, v1, v2, cst)


def kernel(x, edge_index, batch, W1a, b1a, g1a, be1a, W2a, b2a, g2a, be2a,
           W1b, b1b, g1b, be1b, W2b, b2b, g2b, be2b, gbn, bbn, Wl, bl):
    f32 = jnp.float32
    src = edge_index[0]
    dst = edge_index[1]
    pk = jnp.bitwise_or(src, jnp.left_shift(dst, 14))
    pk_pad = jnp.left_shift(N + (jnp.arange(PAD_E, dtype=jnp.int32) % 16), 14)
    packed = jnp.concatenate([pk, pk_pad]).reshape(NW, NCH, CH)
    zeros = jnp.zeros((STRIPE, D), f32)
    batch3 = batch.reshape(NBLK, 1, BLK)

    sc = 1.0 / jnp.sqrt(1.0 + BN_EPS)
    w2a_f = (g1a * sc)[:, None] * W2a
    b2a_f = (b2a + be1a @ W2a).reshape(1, D)
    s2a = (g2a * sc).reshape(1, D)
    t2a = be2a.reshape(1, D)
    w2b_f = (g1b * sc)[:, None] * W2b
    b2b_f = (b2b + be1b @ W2b).reshape(1, D)
    s2b = (g2b * sc).reshape(1, D)
    t2b = be2b.reshape(1, D)
    wl = Wl[:, 0]
    v = (gbn * sc) * wl
    v0 = v[0:D].reshape(1, D)
    v1 = v[D:2 * D].reshape(1, D)
    v2 = v[2 * D:3 * D].reshape(1, D)
    cst = jnp.full((1, D), bl[0] + bbn @ wl, f32)

    agg1 = _sc_segment_sum(x, packed, zeros)
    h1, p0, p1 = _tc1(x, agg1[0, :N], agg1[1, :N], batch3,
                      W1a, b1a.reshape(1, D), w2a_f, b2a_f, s2a, t2a)
    agg2 = _sc_segment_sum(h1, packed, zeros)
    outb = _tc2(h1, agg2[0, :N], agg2[1, :N], batch3,
                W1b, b1b.reshape(1, D), w2b_f, b2b_f, s2b, t2b,
                p0, p1, v0, v1, v2, cst)
    return outb[0][:, 0]
